# Initial kernel scaffold; baseline (speedup 1.0000x reference)
import jax
import jax.numpy as jnp
from jax.experimental import pallas as pl


def kernel(x, W1, b1, W2, b2, fc_linear, protos, ex2, ex1, cls_num):
    raise NotImplementedError("write your pallas kernel here")


# speedup vs baseline: 27.1675x; 27.1675x over previous
import jax
import jax.numpy as jnp
from jax.experimental import pallas as pl

_B = 512
_D = 128
_K = 1000

_DN = (((1,), (1,)), ((), ()))
_PREC = jax.lax.Precision.HIGHEST


def _first_index_of(vals, target, axis):
    ii = jax.lax.broadcasted_iota(jnp.int32, vals.shape, axis)
    return jnp.min(jnp.where(vals == target, ii, vals.shape[axis]), axis=axis)


def _fused(x_ref, w1_ref, b1_ref, w2_ref, b2_ref, fc_ref, pr_ref,
           ex2_ref, ex1_ref, n_ref, p1_ref, p2_ref):
    x = x_ref[...]

    h = jnp.maximum(
        jnp.dot(x, w1_ref[...], preferred_element_type=jnp.float32)
        + b1_ref[...], 0.0)
    m = (jnp.dot(h, w2_ref[...], preferred_element_type=jnp.float32)
         + b2_ref[...])
    nrm = jnp.sqrt(jnp.sum(m * m))
    m = jnp.where(nrm == 0.0, m, m / nrm)
    logits = jax.lax.dot_general(m, fc_ref[...], _DN,
                                 preferred_element_type=jnp.float32)
    mx = jnp.max(logits, axis=1, keepdims=True)
    p1_ref[...] = _first_index_of(logits, mx, axis=1)[None, :]

    n = n_ref[...]
    ex2 = ex2_ref[...]
    ex1 = ex1_ref[...]
    rdr = jnp.sqrt(n * ex2 * ex2 - ex1 * ex1)
    z = jnp.max(rdr, axis=1, keepdims=True) - rdr
    e = jnp.exp(z - jnp.max(z, axis=1, keepdims=True))
    rd = e / jnp.sum(e, axis=1, keepdims=True)

    pr = pr_ref[...]
    w = pr * rd
    c = jnp.sum(pr * w, axis=1)
    simi = (jax.lax.dot_general(x * x, rd, _DN,
                                preferred_element_type=jnp.float32,
                                precision=_PREC)
            - 2.0 * jax.lax.dot_general(x, w, _DN,
                                        preferred_element_type=jnp.float32,
                                        precision=_PREC)
            + c[None, :])
    mn = jnp.min(simi, axis=1, keepdims=True)
    p2_ref[...] = _first_index_of(simi, mn, axis=1)[None, :]


def kernel(x, W1, b1, W2, b2, fc_linear, protos, ex2, ex1, cls_num):
    nf = cls_num.astype(jnp.float32)[:, None]
    p1, p2 = pl.pallas_call(
        _fused,
        out_shape=(
            jax.ShapeDtypeStruct((1, _B), jnp.int32),
            jax.ShapeDtypeStruct((1, _B), jnp.int32),
        ),
    )(x, W1, b1[None, :], W2, b2[None, :], fc_linear, protos,
      ex2, ex1, nf)
    return p1[0], p2[0]


# speedup vs baseline: 31.4995x; 1.1595x over previous
import jax
import jax.numpy as jnp
from jax.experimental import pallas as pl

_B = 512
_D = 128
_K = 1000

_DN = (((1,), (1,)), ((), ()))
_PREC = jax.lax.Precision.HIGHEST


def _first_index_of(vals, target, axis):
    ii = jax.lax.broadcasted_iota(jnp.int32, vals.shape, axis)
    return jnp.min(jnp.where(vals == target, ii, vals.shape[axis]), axis=axis)


def _fused(x_ref, w1_ref, b1_ref, w2_ref, b2_ref, fc_ref, pr_ref,
           ex2_ref, ex1_ref, n_ref, p1_ref, p2_ref):
    x = x_ref[...]

    h = jnp.maximum(
        jnp.dot(x, w1_ref[...], preferred_element_type=jnp.float32)
        + b1_ref[...], 0.0)
    m = (jnp.dot(h, w2_ref[...], preferred_element_type=jnp.float32)
         + b2_ref[...])
    nrm = jnp.sqrt(jnp.sum(m * m))
    m = jnp.where(nrm == 0.0, m, m / nrm)
    logits = jax.lax.dot_general(m, fc_ref[...], _DN,
                                 preferred_element_type=jnp.float32)
    mx = jnp.max(logits, axis=1, keepdims=True)
    p1_ref[...] = _first_index_of(logits, mx, axis=1)[None, :]

    n = n_ref[...]
    ex2 = ex2_ref[...]
    ex1 = ex1_ref[...]
    rdr = jnp.sqrt(n * ex2 * ex2 - ex1 * ex1)
    z = jnp.max(rdr, axis=1, keepdims=True) - rdr
    e = jnp.exp(z - jnp.max(z, axis=1, keepdims=True))
    rd = e / jnp.sum(e, axis=1, keepdims=True)

    pr = pr_ref[...]
    w = pr * rd
    c = jnp.sum(pr * w, axis=1)
    lhs = jnp.concatenate([x * x, x], axis=1)
    rhs = jnp.concatenate([rd, -2.0 * w], axis=1)
    simi = (jax.lax.dot_general(lhs, rhs, _DN,
                                preferred_element_type=jnp.float32,
                                precision=_PREC)
            + c[None, :])
    mn = jnp.min(simi, axis=1, keepdims=True)
    p2_ref[...] = _first_index_of(simi, mn, axis=1)[None, :]


def kernel(x, W1, b1, W2, b2, fc_linear, protos, ex2, ex1, cls_num):
    nf = cls_num.astype(jnp.float32)[:, None]
    p1, p2 = pl.pallas_call(
        _fused,
        out_shape=(
            jax.ShapeDtypeStruct((1, _B), jnp.int32),
            jax.ShapeDtypeStruct((1, _B), jnp.int32),
        ),
    )(x, W1, b1[None, :], W2, b2[None, :], fc_linear, protos,
      ex2, ex1, nf)
    return p1[0], p2[0]
